# Initial kernel scaffold; baseline (speedup 1.0000x reference)
import functools

import jax
import jax.numpy as jnp
from jax import lax
from jax.experimental import pallas as pl
from jax.experimental.pallas import tpu as pltpu
from jax.experimental.pallas import tpu_sc as plsc

_NC = 2
_NS = 16
_NW = _NC * _NS
_L = 16


def _make_sc_partials(n_neurons, n_edges, batch, chunk, interpret=False):
    slices = _NW // batch
    epw = n_edges // slices
    chunks_pw = epw // chunk
    assert epw * slices == n_edges and chunks_pw * chunk == epw
    assert chunk % _L == 0 and chunk % 8 == 0

    mesh = plsc.VectorSubcoreMesh(
        core_axis_name="c", subcore_axis_name="s",
        num_cores=_NC, num_subcores=_NS)

    @functools.partial(
        pl.kernel,
        out_type=jax.ShapeDtypeStruct((_NW, n_neurons), jnp.float32),
        mesh=mesh,
        scratch_types=[
            pltpu.VMEM((n_neurons,), jnp.float32),
            pltpu.VMEM((n_neurons,), jnp.float32),
            pltpu.VMEM((2, chunk), jnp.int32),
            pltpu.VMEM((2, chunk), jnp.int32),
            pltpu.VMEM((2, chunk), jnp.float32),
            pltpu.SemaphoreType.DMA,
            pltpu.SemaphoreType.DMA,
            pltpu.SemaphoreType.DMA,
        ],
        interpret=interpret,
    )
    def sc_partials(z_hbm, edge_hbm, w_hbm, part_hbm,
                    z_v, acc_v, pre_v, post_v, w_v, sem0, sem1, semz):
        wid = lax.axis_index("s") * _NC + lax.axis_index("c")
        b = wid % batch
        s = wid // batch
        zcopy = pltpu.async_copy(z_hbm.at[b], z_v, semz)

        zero = jnp.zeros((_L,), jnp.float32)

        def zbody(i, carry):
            acc_v[pl.ds(i * _L, _L)] = zero
            return carry
        lax.fori_loop(0, n_neurons // _L, zbody, 0, unroll=8)
        zcopy.wait()

        base = s * chunks_pw
        sems = (sem0, sem1)

        def start(g, slot):
            off = (base + g) * chunk
            pltpu.async_copy(edge_hbm.at[1, pl.ds(off, chunk)],
                             pre_v.at[slot], sems[slot])
            pltpu.async_copy(edge_hbm.at[0, pl.ds(off, chunk)],
                             post_v.at[slot], sems[slot])
            pltpu.async_copy(w_hbm.at[pl.ds(off, chunk)],
                             w_v.at[slot], sems[slot])

        def drain(g, slot):
            off = (base + g) * chunk
            pltpu.make_async_copy(edge_hbm.at[1, pl.ds(off, chunk)],
                                  pre_v.at[slot], sems[slot]).wait()
            pltpu.make_async_copy(edge_hbm.at[0, pl.ds(off, chunk)],
                                  post_v.at[slot], sems[slot]).wait()
            pltpu.make_async_copy(w_hbm.at[pl.ds(off, chunk)],
                                  w_v.at[slot], sems[slot]).wait()

        start(0, 0)
        start(1, 1)

        def pair_body(gp, carry):
            for slot in range(2):
                g = gp * 2 + slot
                drain(g, slot)

                def inner(j, c2):
                    sl = pl.ds(j * _L, _L)
                    pre = pre_v[slot, sl]
                    post = post_v[slot, sl]
                    wv = w_v[slot, sl]
                    zg = plsc.load_gather(z_v, [pre])
                    plsc.addupdate_scatter(acc_v, [post], zg * wv)
                    return c2
                lax.fori_loop(0, chunk // _L, inner, 0, unroll=4)

                @pl.when(g + 2 < chunks_pw)
                def _():
                    start(g + 2, slot)
            return carry
        lax.fori_loop(0, chunks_pw // 2, pair_body, 0)

        pltpu.sync_copy(acc_v, part_hbm.at[wid])

    return sc_partials


def _lif_body(p_ref, v_ref, decay_ref, cf_ref, vth_ref, vreset_ref,
              z_out, v_out):
    rec = jnp.sum(p_ref[...], axis=0)
    v = v_ref[...]
    decay = decay_ref[...]
    cf = cf_ref[...]
    vth = vth_ref[...]
    vreset = vreset_ref[...]
    new_v = decay * v + cf * rec
    v_scaled = (new_v - vth) / jnp.maximum(vth - vreset, 1e-6)
    new_z = (v_scaled > 0.0).astype(jnp.float32)
    z_out[...] = new_z
    v_out[...] = new_v - new_z * (vth - vreset)


def kernel(z, v, edge_index, weights, decay, current_factor, v_th, v_reset):
    batch, n = z.shape
    n_edges = weights.shape[0]
    chunk = 2000

    sc = _make_sc_partials(n, n_edges, batch, chunk)
    partials = sc(z, edge_index, weights)
    partials = partials.reshape(_NW // batch, batch, n)

    d2 = decay.reshape(1, n)
    cf2 = current_factor.reshape(1, n)
    vth2 = v_th.reshape(1, n)
    vr2 = v_reset.reshape(1, n)
    new_z, reset_v = pl.pallas_call(
        _lif_body,
        out_shape=(jax.ShapeDtypeStruct((batch, n), jnp.float32),
                   jax.ShapeDtypeStruct((batch, n), jnp.float32)),
    )(partials, v, d2, cf2, vth2, vr2)
    return jnp.stack([new_z, reset_v], axis=0)


# speedup vs baseline: 39.3438x; 39.3438x over previous
import functools

import jax
import jax.numpy as jnp
from jax import lax
from jax.experimental import pallas as pl
from jax.experimental.pallas import tpu as pltpu
from jax.experimental.pallas import tpu_sc as plsc

_NC = 2
_NS = 16
_NW = _NC * _NS
_L = 16


def _make_sc_partials(n_neurons, n_edges, batch, chunk):
    slices = _NW // batch
    epw = n_edges // slices
    chunks_pw = epw // chunk
    assert epw * slices == n_edges and chunks_pw * chunk == epw
    assert chunk % _L == 0 and chunk % 8 == 0 and chunks_pw % 2 == 0

    mesh = plsc.VectorSubcoreMesh(
        core_axis_name="c", subcore_axis_name="s",
        num_cores=_NC, num_subcores=_NS)

    @functools.partial(
        pl.kernel,
        out_type=jax.ShapeDtypeStruct((_NW * n_neurons,), jnp.float32),
        mesh=mesh,
        scratch_types=[
            pltpu.VMEM((n_neurons,), jnp.float32),
            pltpu.VMEM((n_neurons,), jnp.float32),
            pltpu.VMEM((chunk,), jnp.int32),
            pltpu.VMEM((chunk,), jnp.int32),
            pltpu.VMEM((chunk,), jnp.int32),
            pltpu.VMEM((chunk,), jnp.int32),
            pltpu.VMEM((chunk,), jnp.float32),
            pltpu.VMEM((chunk,), jnp.float32),
            pltpu.SemaphoreType.DMA,
            pltpu.SemaphoreType.DMA,
            pltpu.SemaphoreType.DMA,
        ],
        compiler_params=pltpu.CompilerParams(needs_layout_passes=False),
    )
    def sc_partials(z_hbm, pre_hbm, post_hbm, w_hbm, part_hbm,
                    z_v, acc_v, pre0, pre1, post0, post1, w0, w1,
                    sem0, sem1, semz):
        wid = lax.axis_index("s") * _NC + lax.axis_index("c")
        b = wid % batch
        s = wid // batch
        zcopy = pltpu.async_copy(
            z_hbm.at[pl.ds(b * n_neurons, n_neurons)], z_v, semz)

        zero = jnp.zeros((_L,), jnp.float32)

        def zbody(i, carry):
            acc_v[pl.ds(i * _L, _L)] = zero
            return carry
        lax.fori_loop(0, n_neurons // _L, zbody, 0, unroll=8)
        zcopy.wait()

        base = s * chunks_pw
        bufs = ((pre0, post0, w0), (pre1, post1, w1))
        sems = (sem0, sem1)

        def start(g, slot):
            off = (base + g) * chunk
            pv, qv, wv = bufs[slot]
            pltpu.async_copy(pre_hbm.at[pl.ds(off, chunk)], pv, sems[slot])
            pltpu.async_copy(post_hbm.at[pl.ds(off, chunk)], qv, sems[slot])
            pltpu.async_copy(w_hbm.at[pl.ds(off, chunk)], wv, sems[slot])

        def drain(g, slot):
            off = (base + g) * chunk
            pv, qv, wv = bufs[slot]
            pltpu.make_async_copy(pre_hbm.at[pl.ds(off, chunk)], pv,
                                  sems[slot]).wait()
            pltpu.make_async_copy(post_hbm.at[pl.ds(off, chunk)], qv,
                                  sems[slot]).wait()
            pltpu.make_async_copy(w_hbm.at[pl.ds(off, chunk)], wv,
                                  sems[slot]).wait()

        start(0, 0)
        start(1, 1)

        def pair_body(gp, carry):
            for slot in range(2):
                g = gp * 2 + slot
                drain(g, slot)
                pv, qv, wv = bufs[slot]

                def inner(j, c2):
                    sl = pl.ds(j * _L, _L)
                    pre = pv[sl]
                    post = qv[sl]
                    w = wv[sl]
                    zg = plsc.load_gather(z_v, [pre])
                    plsc.addupdate_scatter(acc_v, [post], zg * w)
                    return c2
                lax.fori_loop(0, chunk // _L, inner, 0, unroll=4)

                @pl.when(g + 2 < chunks_pw)
                def _():
                    start(g + 2, slot)
            return carry
        lax.fori_loop(0, chunks_pw // 2, pair_body, 0)

        pltpu.sync_copy(acc_v, part_hbm.at[pl.ds(wid * n_neurons, n_neurons)])

    return sc_partials


def _lif_body(p_ref, v_ref, decay_ref, cf_ref, vth_ref, vreset_ref,
              z_out, v_out):
    rec = jnp.sum(p_ref[...], axis=0)
    v = v_ref[...]
    decay = decay_ref[...]
    cf = cf_ref[...]
    vth = vth_ref[...]
    vreset = vreset_ref[...]
    new_v = decay * v + cf * rec
    v_scaled = (new_v - vth) / jnp.maximum(vth - vreset, 1e-6)
    new_z = (v_scaled > 0.0).astype(jnp.float32)
    z_out[...] = new_z
    v_out[...] = new_v - new_z * (vth - vreset)


def kernel(z, v, edge_index, weights, decay, current_factor, v_th, v_reset):
    batch, n = z.shape
    n_edges = weights.shape[0]
    chunk = 2000

    sc = _make_sc_partials(n, n_edges, batch, chunk)
    partials = sc(z.reshape(-1), edge_index[1], edge_index[0], weights)
    partials = partials.reshape(_NW // batch, batch, n)

    d2 = decay.reshape(1, n)
    cf2 = current_factor.reshape(1, n)
    vth2 = v_th.reshape(1, n)
    vr2 = v_reset.reshape(1, n)
    new_z, reset_v = pl.pallas_call(
        _lif_body,
        out_shape=(jax.ShapeDtypeStruct((batch, n), jnp.float32),
                   jax.ShapeDtypeStruct((batch, n), jnp.float32)),
    )(partials, v, d2, cf2, vth2, vr2)
    return jnp.stack([new_z, reset_v], axis=0)


# speedup vs baseline: 69.2555x; 1.7603x over previous
import functools

import jax
import jax.numpy as jnp
from jax import lax
from jax.experimental import pallas as pl
from jax.experimental.pallas import tpu as pltpu
from jax.experimental.pallas import tpu_sc as plsc

_NC = 2
_NS = 16
_NW = _NC * _NS
_L = 16


def _make_sc_partials(n_neurons, n_edges, batch, chunk):
    slices = _NW // batch
    epw = n_edges // slices
    chunks_pw = epw // chunk
    assert epw * slices == n_edges and chunks_pw * chunk == epw
    assert chunk % _L == 0 and chunk % 8 == 0 and chunks_pw % 2 == 0

    mesh = plsc.VectorSubcoreMesh(
        core_axis_name="c", subcore_axis_name="s",
        num_cores=_NC, num_subcores=_NS)

    @functools.partial(
        pl.kernel,
        out_type=jax.ShapeDtypeStruct((_NW * n_neurons,), jnp.float32),
        mesh=mesh,
        scratch_types=[
            pltpu.VMEM((n_neurons,), jnp.float32),
            pltpu.VMEM((n_neurons,), jnp.float32),
            pltpu.VMEM((chunk,), jnp.int32),
            pltpu.VMEM((chunk,), jnp.int32),
            pltpu.VMEM((chunk,), jnp.int32),
            pltpu.VMEM((chunk,), jnp.int32),
            pltpu.VMEM((chunk,), jnp.float32),
            pltpu.VMEM((chunk,), jnp.float32),
            pltpu.SemaphoreType.DMA,
            pltpu.SemaphoreType.DMA,
            pltpu.SemaphoreType.DMA,
        ],
        compiler_params=pltpu.CompilerParams(needs_layout_passes=False),
    )
    def sc_partials(z_hbm, edge_hbm, w_hbm, part_hbm,
                    z_v, acc_v, pre0, pre1, post0, post1, w0, w1,
                    sem0, sem1, semz):
        wid = lax.axis_index("s") * _NC + lax.axis_index("c")
        b = wid % batch
        s = wid // batch
        zcopy = pltpu.async_copy(
            z_hbm.at[pl.ds(b * n_neurons, n_neurons)], z_v, semz)

        zero = jnp.zeros((_L,), jnp.float32)

        def zbody(i, carry):
            acc_v[pl.ds(i * _L, _L)] = zero
            return carry
        lax.fori_loop(0, n_neurons // _L, zbody, 0, unroll=8)
        zcopy.wait()

        base = s * chunks_pw
        bufs = ((pre0, post0, w0), (pre1, post1, w1))
        sems = (sem0, sem1)

        def start(g, slot):
            off = (base + g) * chunk
            pv, qv, wv = bufs[slot]
            pltpu.async_copy(edge_hbm.at[pl.ds(n_edges + off, chunk)], pv,
                             sems[slot])
            pltpu.async_copy(edge_hbm.at[pl.ds(off, chunk)], qv, sems[slot])
            pltpu.async_copy(w_hbm.at[pl.ds(off, chunk)], wv, sems[slot])

        def drain(g, slot):
            off = (base + g) * chunk
            pv, qv, wv = bufs[slot]
            pltpu.make_async_copy(edge_hbm.at[pl.ds(n_edges + off, chunk)],
                                  pv, sems[slot]).wait()
            pltpu.make_async_copy(edge_hbm.at[pl.ds(off, chunk)], qv,
                                  sems[slot]).wait()
            pltpu.make_async_copy(w_hbm.at[pl.ds(off, chunk)], wv,
                                  sems[slot]).wait()

        start(0, 0)
        start(1, 1)

        def pair_body(gp, carry):
            for slot in range(2):
                g = gp * 2 + slot
                drain(g, slot)
                pv, qv, wv = bufs[slot]

                @plsc.parallel_loop(0, chunk // _L, unroll=8)
                def _(j):
                    sl = pl.ds(j * _L, _L)
                    pre = pv[sl]
                    post = qv[sl]
                    w = wv[sl]
                    zg = plsc.load_gather(z_v, [pre])
                    plsc.addupdate_scatter(acc_v, [post], zg * w)

                @pl.when(g + 2 < chunks_pw)
                def _():
                    start(g + 2, slot)
            return carry
        lax.fori_loop(0, chunks_pw // 2, pair_body, 0)

        pltpu.sync_copy(acc_v, part_hbm.at[pl.ds(wid * n_neurons, n_neurons)])

    return sc_partials


def _lif_body(p_ref, v_ref, decay_ref, cf_ref, vth_ref, vreset_ref, out_ref):
    rec = jnp.sum(p_ref[...], axis=0)
    v = v_ref[...]
    decay = decay_ref[...]
    cf = cf_ref[...]
    vth = vth_ref[...]
    vreset = vreset_ref[...]
    new_v = decay * v + cf * rec
    v_scaled = (new_v - vth) / jnp.maximum(vth - vreset, 1e-6)
    new_z = (v_scaled > 0.0).astype(jnp.float32)
    out_ref[0] = new_z
    out_ref[1] = new_v - new_z * (vth - vreset)


def kernel(z, v, edge_index, weights, decay, current_factor, v_th, v_reset):
    batch, n = z.shape
    n_edges = weights.shape[0]
    chunk = 2000

    sc = _make_sc_partials(n, n_edges, batch, chunk)
    partials = sc(z.reshape(-1), edge_index.reshape(-1), weights)
    partials = partials.reshape(_NW // batch, batch, n)

    d2 = decay.reshape(1, n)
    cf2 = current_factor.reshape(1, n)
    vth2 = v_th.reshape(1, n)
    vr2 = v_reset.reshape(1, n)
    return pl.pallas_call(
        _lif_body,
        out_shape=jax.ShapeDtypeStruct((2, batch, n), jnp.float32),
    )(partials, v, d2, cf2, vth2, vr2)
